# baseline (speedup 1.0000x reference)
import functools
import math

import jax
import jax.numpy as jnp
from jax import lax
from jax.experimental import pallas as pl
from jax.experimental.pallas import tpu as pltpu
from jax.experimental.pallas import tpu_sc as plsc

N = 10000
E = 320000
HIDDEN = 128
NUM_RADIAL = 16
CUTOFF = 5.0

NC = 2
NS = 16
NW = NC * NS

CHUNK = 128
NCHUNKS = E // CHUNK
ROWS_PER_SUB = N // NS

BE = 3200

_PREC = lax.Precision.DEFAULT


def _silu(v):
    return v * (1.0 / (1.0 + jnp.exp(-v)))


def _precompute_body(x_ref, wa_ref, wb_ref, xa_ref, xb_ref):
    x = x_ref[...]
    xa_ref[...] = lax.dot_general(x, wa_ref[...], (((1,), (0,)), ((), ())),
                                  preferred_element_type=jnp.float32, precision=_PREC)
    xb_ref[...] = lax.dot_general(x, wb_ref[...], (((1,), (0,)), ((), ())),
                                  preferred_element_type=jnp.float32, precision=_PREC)


def _precompute(x, wa, wb):
    return pl.pallas_call(
        _precompute_body,
        out_shape=[jax.ShapeDtypeStruct((N, HIDDEN), jnp.float32),
                   jax.ShapeDtypeStruct((N, HIDDEN), jnp.float32)],
    )(x, wa, wb)


def _sc_gather(xa, xb, small, src, dst, e0, ne):
    mesh = plsc.VectorSubcoreMesh(core_axis_name="c", subcore_axis_name="s")
    nchunks = ne // CHUNK
    npairs = ((nchunks + NW - 1) // NW + 1) // 2

    buf2 = lambda shape, dt: [pltpu.VMEM(shape, dt), pltpu.VMEM(shape, dt)]

    @functools.partial(
        pl.kernel,
        mesh=mesh,
        compiler_params=pltpu.CompilerParams(use_tc_tiling_on_sc=False),
        out_type=[jax.ShapeDtypeStruct((ne, HIDDEN), jnp.float32),
                  jax.ShapeDtypeStruct((ne, HIDDEN), jnp.float32),
                  jax.ShapeDtypeStruct((ne, 16), jnp.float32),
                  jax.ShapeDtypeStruct((ne, 16), jnp.float32)],
        scratch_types=(buf2((CHUNK,), jnp.int32) + buf2((CHUNK,), jnp.int32)
                       + buf2((CHUNK, HIDDEN), jnp.float32)
                       + buf2((CHUNK, HIDDEN), jnp.float32)
                       + buf2((CHUNK, 16), jnp.float32)
                       + buf2((CHUNK, 16), jnp.float32)
                       + [pltpu.SemaphoreType.DMA, pltpu.SemaphoreType.DMA]),
    )
    def k(xa_hbm, xb_hbm, small_hbm, src_hbm, dst_hbm,
          oa_hbm, ob_hbm, ops_hbm, opd_hbm,
          src_v0, src_v1, dst_v0, dst_v1, bufa0, bufa1, bufb0, bufb1,
          bufps0, bufps1, bufpd0, bufpd1, semg, semw):
        wid = lax.axis_index("s") * NC + lax.axis_index("c")
        src_v = (src_v0, src_v1)
        dst_v = (dst_v0, dst_v1)
        bufa = (bufa0, bufa1)
        bufb = (bufb0, bufb1)
        bufps = (bufps0, bufps1)
        bufpd = (bufpd0, bufpd1)

        @pl.loop(0, npairs)
        def _(t):
            bases = []
            gathers = []
            for p in range(2):
                c = (2 * t + p) * NW + wid
                c = jnp.where(c < nchunks, c, c - nchunks)
                base = c * CHUNK
                bases.append(base)
                pltpu.sync_copy(src_hbm.at[pl.ds(e0 + base, CHUNK)], src_v[p])
                pltpu.sync_copy(dst_hbm.at[pl.ds(e0 + base, CHUNK)], dst_v[p])
                gathers.append([
                    pltpu.async_copy(xa_hbm.at[src_v[p]], bufa[p], semg),
                    pltpu.async_copy(xb_hbm.at[dst_v[p]], bufb[p], semg),
                    pltpu.async_copy(small_hbm.at[src_v[p]], bufps[p], semg),
                    pltpu.async_copy(small_hbm.at[dst_v[p]], bufpd[p], semg),
                ])
            writes = []
            for p in range(2):
                for h in gathers[p]:
                    h.wait()
                writes += [
                    pltpu.async_copy(bufa[p], oa_hbm.at[pl.ds(bases[p], CHUNK)], semw),
                    pltpu.async_copy(bufb[p], ob_hbm.at[pl.ds(bases[p], CHUNK)], semw),
                    pltpu.async_copy(bufps[p], ops_hbm.at[pl.ds(bases[p], CHUNK)], semw),
                    pltpu.async_copy(bufpd[p], opd_hbm.at[pl.ds(bases[p], CHUNK)], semw),
                ]
            for h in writes:
                h.wait()

    return k(xa, xb, small, src, dst)


def _edge_body(xa_ref, xb_ref, ps_ref, pd_ref, attr_ref,
               we1s_ref, be1_ref, lng_ref, lnb_ref, we2_ref, be2_ref,
               wc1_ref, bc1_ref, wc2_ref, bc2_ref,
               hid_ref, aux_ref):
    psT = ps_ref[...].T
    pdT = pd_ref[...].T
    attrT = attr_ref[...].T

    relT = pdT[0:3] - psT[0:3]
    dist2T = relT[0:1] * relT[0:1] + relT[1:2] * relT[1:2] + relT[2:3] * relT[2:3]
    distT = jnp.sqrt(dist2T + 1e-8)
    safeT = jnp.maximum(distT, 1e-6)
    unitT = relT / safeT

    def leg_score(row):
        ca = jnp.cos(attrT[row:row + 1])
        p0 = jnp.ones_like(ca)
        p1 = ca
        p2 = (3.0 * ca * p1 - p0) * (1.0 / 2.0)
        p3 = (5.0 * ca * p2 - 2.0 * p1) * (1.0 / 3.0)
        return (jnp.abs(p0) + jnp.abs(p1) + jnp.abs(p2) + jnp.abs(p3)) * 0.25

    angle_score = leg_score(0) * attrT[2:3]
    dihedral_score = leg_score(1) * attrT[3:4]
    gateT = jnp.clip(1.0 + 0.6 * (angle_score + dihedral_score), 0.35, 2.5)

    theta = (math.pi / CUTOFF) * safeT
    s1 = jnp.sin(theta)
    c2 = 2.0 * jnp.cos(theta)
    inv_clip = 1.0 / safeT
    sines = [s1]
    prev, cur = jnp.zeros_like(s1), s1
    for _ in range(NUM_RADIAL - 1):
        prev, cur = cur, c2 * cur - prev
        sines.append(cur)

    csT = psT[3:4]
    cdT = pdT[3:4]
    scaled_gate = gateT * inv_clip
    featT = jnp.concatenate(
        [s * scaled_gate for s in sines] + [
            distT * (1.0 / CUTOFF) * gateT,
            csT * cdT * gateT,
            jnp.abs(csT - cdT) * gateT,
            jnp.zeros((13,) + gateT.shape[1:], jnp.float32),
        ], axis=0)

    pre = xa_ref[...] + xb_ref[...]
    pre = pre + lax.dot_general(featT, we1s_ref[...], (((0,), (0,)), ((), ())),
                                preferred_element_type=jnp.float32, precision=_PREC)
    pre = pre + be1_ref[...]
    h = _silu(pre)
    mu = jnp.mean(h, axis=1, keepdims=True)
    var = jnp.mean((h - mu) ** 2, axis=1, keepdims=True)
    h = (h - mu) / jnp.sqrt(var + 1e-5) * lng_ref[...] + lnb_ref[...]
    h2 = lax.dot_general(h, we2_ref[...], (((1,), (0,)), ((), ())),
                         preferred_element_type=jnp.float32, precision=_PREC)
    hid = _silu(h2 + be2_ref[...])
    hid_ref[...] = hid

    t = _silu(lax.dot_general(hid, wc1_ref[...], (((1,), (0,)), ((), ())),
                              preferred_element_type=jnp.float32, precision=_PREC)
              + bc1_ref[...])
    coord_w = jnp.sum(t * wc2_ref[...], axis=1, keepdims=True) + bc2_ref[...]
    wvecT = unitT * (coord_w.T * gateT)
    auxT = jnp.concatenate(
        [jnp.ones_like(gateT), wvecT,
         jnp.zeros((12,) + gateT.shape[1:], jnp.float32)], axis=0)
    aux_ref[...] = auxT.T


def _edge_mlp(xa_g, xb_g, ps_g, pd_g, edge_attr, we1s, be1, lng, lnb, we2, be2,
              wc1, bc1, wc2row, bc2, e0, ne):
    nblocks = ne // BE
    blk0 = e0 // BE
    full = lambda shape: pl.BlockSpec(shape, lambda i: (0, 0))
    return pl.pallas_call(
        _edge_body,
        grid=(nblocks,),
        in_specs=[
            pl.BlockSpec((BE, HIDDEN), lambda i: (i, 0)),
            pl.BlockSpec((BE, HIDDEN), lambda i: (i, 0)),
            pl.BlockSpec((BE, 16), lambda i: (i, 0)),
            pl.BlockSpec((BE, 16), lambda i: (i, 0)),
            pl.BlockSpec((BE, 4), lambda i: (i + blk0, 0)),
            full((32, HIDDEN)),
            full((1, HIDDEN)),
            full((1, HIDDEN)),
            full((1, HIDDEN)),
            full((HIDDEN, HIDDEN)),
            full((1, HIDDEN)),
            full((HIDDEN, 64)),
            full((1, 64)),
            full((1, 64)),
            full((1, 1)),
        ],
        out_specs=[
            pl.BlockSpec((BE, HIDDEN), lambda i: (i, 0)),
            pl.BlockSpec((BE, 16), lambda i: (i, 0)),
        ],
        out_shape=[jax.ShapeDtypeStruct((ne, HIDDEN), jnp.float32),
                   jax.ShapeDtypeStruct((ne, 16), jnp.float32)],
    )(xa_g, xb_g, ps_g, pd_g, edge_attr, we1s, be1, lng, lnb, we2, be2,
      wc1, bc1, wc2row, bc2)


def _sc_scatter(hid, aux, src, zeros_h, zeros_a, e0, ne):
    mesh = plsc.VectorSubcoreMesh(core_axis_name="c", subcore_axis_name="s")
    nchunks = ne // CHUNK
    nloop = (nchunks + NW - 1) // NW

    @functools.partial(
        pl.kernel,
        mesh=mesh,
        compiler_params=pltpu.CompilerParams(use_tc_tiling_on_sc=False),
        out_type=[jax.ShapeDtypeStruct((NC, N, HIDDEN), jnp.float32),
                  jax.ShapeDtypeStruct((NC, N, 16), jnp.float32)],
        scratch_types=[pltpu.VMEM((CHUNK,), jnp.int32),
                       pltpu.VMEM((CHUNK, HIDDEN), jnp.float32),
                       pltpu.VMEM((CHUNK, 16), jnp.float32),
                       pltpu.VMEM_SHARED((N, HIDDEN), jnp.float32),
                       pltpu.VMEM_SHARED((N, 16), jnp.float32),
                       pltpu.SemaphoreType.DMA],
    )
    def k(hid_hbm, aux_hbm, src_hbm, zh_hbm, za_hbm, oh_hbm, oa_hbm,
          idx_v, hbuf, abuf, acc_h, acc_a, sem):
        c = lax.axis_index("c")
        s = lax.axis_index("s")

        row0 = s * ROWS_PER_SUB
        pltpu.sync_copy(zh_hbm, acc_h.at[pl.ds(row0, ROWS_PER_SUB)])
        pltpu.sync_copy(za_hbm, acc_a.at[pl.ds(row0, ROWS_PER_SUB)])
        plsc.subcore_barrier()

        @pl.loop(0, nloop)
        def _(j):
            cc = (j * NS + s) * NC + c

            @pl.when(cc < nchunks)
            def _():
                base = cc * CHUNK
                pltpu.sync_copy(src_hbm.at[pl.ds(e0 + base, CHUNK)], idx_v)
                d1 = pltpu.async_copy(hid_hbm.at[pl.ds(base, CHUNK)], hbuf, sem)
                d2 = pltpu.async_copy(aux_hbm.at[pl.ds(base, CHUNK)], abuf, sem)
                d1.wait(); d2.wait()
                pltpu.sync_copy(hbuf, acc_h.at[idx_v], add=True)
                pltpu.sync_copy(abuf, acc_a.at[idx_v], add=True)

        plsc.subcore_barrier()
        pltpu.sync_copy(acc_h.at[pl.ds(row0, ROWS_PER_SUB)],
                        oh_hbm.at[c, pl.ds(row0, ROWS_PER_SUB)])
        pltpu.sync_copy(acc_a.at[pl.ds(row0, ROWS_PER_SUB)],
                        oa_hbm.at[c, pl.ds(row0, ROWS_PER_SUB)])

    return k(hid, aux, src, zeros_h, zeros_a)


def _node_body(nparts, x_ref, pos_ref, *refs):
    (ph_refs, pa_refs) = refs[:nparts], refs[nparts:2 * nparts]
    (wn1_ref, wn2_ref, bn_ref, lng_ref, lnb_ref, xo_ref, po_ref) = refs[2 * nparts:]
    x = x_ref[...]
    hs = sum(r[0] + r[1] for r in ph_refs)
    as_ = sum(r[0] + r[1] for r in pa_refs)
    cnt = jnp.maximum(as_[:, 0:1], 1.0)
    inv = 1.0 / cnt
    agg = hs * inv
    nh = (lax.dot_general(x, wn1_ref[...], (((1,), (0,)), ((), ())),
                          preferred_element_type=jnp.float32, precision=_PREC)
          + lax.dot_general(agg, wn2_ref[...], (((1,), (0,)), ((), ())),
                            preferred_element_type=jnp.float32, precision=_PREC)
          + bn_ref[...])
    nh = _silu(nh)
    mu = jnp.mean(nh, axis=1, keepdims=True)
    var = jnp.mean((nh - mu) ** 2, axis=1, keepdims=True)
    nh = (nh - mu) / jnp.sqrt(var + 1e-5) * lng_ref[...] + lnb_ref[...]
    xo_ref[...] = x + nh
    delta = as_[:, 1:4] * inv
    po_ref[...] = pos_ref[...] + 0.1 * delta


BN = 2000


def _node_update(x, pos, parts_h, parts_a, wn1, wn2, bn, lng, lnb):
    nparts = len(parts_h)
    full = lambda shape: pl.BlockSpec(shape, lambda i: tuple(0 for _ in shape))
    return pl.pallas_call(
        functools.partial(_node_body, nparts),
        grid=(N // BN,),
        in_specs=[
            pl.BlockSpec((BN, HIDDEN), lambda i: (i, 0)),
            pl.BlockSpec((BN, 3), lambda i: (i, 0)),
        ] + [pl.BlockSpec((NC, BN, HIDDEN), lambda i: (0, i, 0))] * nparts
          + [pl.BlockSpec((NC, BN, 16), lambda i: (0, i, 0))] * nparts
          + [
            full((HIDDEN, HIDDEN)),
            full((HIDDEN, HIDDEN)),
            full((1, HIDDEN)),
            full((1, HIDDEN)),
            full((1, HIDDEN)),
        ],
        out_specs=[
            pl.BlockSpec((BN, HIDDEN), lambda i: (i, 0)),
            pl.BlockSpec((BN, 3), lambda i: (i, 0)),
        ],
        out_shape=[jax.ShapeDtypeStruct((N, HIDDEN), jnp.float32),
                   jax.ShapeDtypeStruct((N, 3), jnp.float32)],
    )(x, pos, *parts_h, *parts_a, wn1, wn2, bn, lng, lnb)


def kernel(x, pos, charge, edge_index, edge_attr, W_e1, b_e1, ln_e_g, ln_e_b,
           W_e2, b_e2, W_n, b_n, ln_n_g, ln_n_b, W_c1, b_c1, W_c2, b_c2):
    src = edge_index[0]
    dst = edge_index[1]

    xa, xb = _precompute(x, W_e1[:HIDDEN], W_e1[HIDDEN:2 * HIDDEN])

    small = jnp.concatenate(
        [pos, charge, jnp.zeros((N, 12), jnp.float32)], axis=1)

    we1s = jnp.pad(W_e1[2 * HIDDEN:], ((0, 32 - (W_e1.shape[0] - 2 * HIDDEN)), (0, 0)))
    zeros_h = jnp.zeros((ROWS_PER_SUB, HIDDEN), jnp.float32)
    zeros_a = jnp.zeros((ROWS_PER_SUB, 16), jnp.float32)

    NSPLIT = 4
    EH = E // NSPLIT
    parts = []
    for i in range(NSPLIT):
        e0 = i * EH
        xa_g, xb_g, ps_g, pd_g = _sc_gather(xa, xb, small, src, dst, e0, EH)
        hid, aux = _edge_mlp(
            xa_g, xb_g, ps_g, pd_g, edge_attr,
            we1s, b_e1.reshape(1, HIDDEN), ln_e_g.reshape(1, HIDDEN),
            ln_e_b.reshape(1, HIDDEN), W_e2, b_e2.reshape(1, HIDDEN),
            W_c1, b_c1.reshape(1, 64), W_c2.reshape(1, 64), b_c2.reshape(1, 1),
            e0, EH)
        parts.append(_sc_scatter(hid, aux, src, zeros_h, zeros_a, e0, EH))

    x_new, pos_new = _node_update(
        x, pos, [p[0] for p in parts], [p[1] for p in parts],
        W_n[:HIDDEN], W_n[HIDDEN:],
        b_n.reshape(1, HIDDEN), ln_n_g.reshape(1, HIDDEN), ln_n_b.reshape(1, HIDDEN))
    return (x_new, pos_new)


# speedup vs baseline: 1.0395x; 1.0395x over previous
import functools
import math

import jax
import jax.numpy as jnp
from jax import lax
from jax.experimental import pallas as pl
from jax.experimental.pallas import tpu as pltpu
from jax.experimental.pallas import tpu_sc as plsc

N = 10000
E = 320000
HIDDEN = 128
NUM_RADIAL = 16
CUTOFF = 5.0

NC = 2
NS = 16
NW = NC * NS

CHUNK = 128
NCHUNKS = E // CHUNK
ROWS_PER_SUB = N // NS

BE = 3200

_PREC = lax.Precision.DEFAULT


def _silu(v):
    return v * (0.5 * jnp.tanh(0.5 * v) + 0.5)


def _precompute_body(x_ref, wa_ref, wb_ref, xa_ref, xb_ref):
    x = x_ref[...]
    xa_ref[...] = lax.dot_general(x, wa_ref[...], (((1,), (0,)), ((), ())),
                                  preferred_element_type=jnp.float32, precision=_PREC)
    xb_ref[...] = lax.dot_general(x, wb_ref[...], (((1,), (0,)), ((), ())),
                                  preferred_element_type=jnp.float32, precision=_PREC)


def _precompute(x, wa, wb):
    return pl.pallas_call(
        _precompute_body,
        out_shape=[jax.ShapeDtypeStruct((N, HIDDEN), jnp.float32),
                   jax.ShapeDtypeStruct((N, HIDDEN), jnp.float32)],
    )(x, wa, wb)


def _sc_gather(xa, xb, small, src, dst, e0, ne):
    mesh = plsc.VectorSubcoreMesh(core_axis_name="c", subcore_axis_name="s")
    nchunks = ne // CHUNK
    nloop = (nchunks + NW - 1) // NW

    @functools.partial(
        pl.kernel,
        mesh=mesh,
        compiler_params=pltpu.CompilerParams(use_tc_tiling_on_sc=False),
        out_type=[jax.ShapeDtypeStruct((ne, HIDDEN), jnp.float32),
                  jax.ShapeDtypeStruct((ne, HIDDEN), jnp.float32),
                  jax.ShapeDtypeStruct((ne, 16), jnp.float32),
                  jax.ShapeDtypeStruct((ne, 16), jnp.float32)],
        scratch_types=[pltpu.VMEM((CHUNK,), jnp.int32),
                       pltpu.VMEM((CHUNK,), jnp.int32),
                       pltpu.VMEM((CHUNK, HIDDEN), jnp.float32),
                       pltpu.VMEM((CHUNK, HIDDEN), jnp.float32),
                       pltpu.VMEM((CHUNK, 16), jnp.float32),
                       pltpu.VMEM((CHUNK, 16), jnp.float32),
                       pltpu.SemaphoreType.DMA],
    )
    def k(xa_hbm, xb_hbm, small_hbm, src_hbm, dst_hbm,
          oa_hbm, ob_hbm, ops_hbm, opd_hbm,
          src_v, dst_v, bufa, bufb, bufps, bufpd, sem):
        wid = lax.axis_index("s") * NC + lax.axis_index("c")

        @pl.loop(0, nloop)
        def _(j):
            c = j * NW + wid

            @pl.when(c < nchunks)
            def _():
                base = c * CHUNK
                pltpu.sync_copy(src_hbm.at[pl.ds(e0 + base, CHUNK)], src_v)
                pltpu.sync_copy(dst_hbm.at[pl.ds(e0 + base, CHUNK)], dst_v)
                d1 = pltpu.async_copy(xa_hbm.at[src_v], bufa, sem)
                d2 = pltpu.async_copy(xb_hbm.at[dst_v], bufb, sem)
                d3 = pltpu.async_copy(small_hbm.at[src_v], bufps, sem)
                d4 = pltpu.async_copy(small_hbm.at[dst_v], bufpd, sem)
                d1.wait(); d2.wait(); d3.wait(); d4.wait()
                pltpu.sync_copy(bufa, oa_hbm.at[pl.ds(base, CHUNK)])
                pltpu.sync_copy(bufb, ob_hbm.at[pl.ds(base, CHUNK)])
                pltpu.sync_copy(bufps, ops_hbm.at[pl.ds(base, CHUNK)])
                pltpu.sync_copy(bufpd, opd_hbm.at[pl.ds(base, CHUNK)])

    return k(xa, xb, small, src, dst)


def _edge_body(xa_ref, xb_ref, ps_ref, pd_ref, attr_ref,
               we1s_ref, be1_ref, lng_ref, lnb_ref, we2_ref, be2_ref,
               wc1_ref, bc1_ref, wc2_ref, bc2_ref,
               hid_ref, aux_ref):
    psT = ps_ref[...].T
    pdT = pd_ref[...].T
    attrT = attr_ref[...].T

    relT = pdT[0:3] - psT[0:3]
    dist2T = relT[0:1] * relT[0:1] + relT[1:2] * relT[1:2] + relT[2:3] * relT[2:3]
    distT = jnp.sqrt(dist2T + 1e-8)
    safeT = jnp.maximum(distT, 1e-6)
    unitT = relT / safeT

    def leg_score(row):
        ca = jnp.cos(attrT[row:row + 1])
        p0 = jnp.ones_like(ca)
        p1 = ca
        p2 = (3.0 * ca * p1 - p0) * (1.0 / 2.0)
        p3 = (5.0 * ca * p2 - 2.0 * p1) * (1.0 / 3.0)
        return (jnp.abs(p0) + jnp.abs(p1) + jnp.abs(p2) + jnp.abs(p3)) * 0.25

    angle_score = leg_score(0) * attrT[2:3]
    dihedral_score = leg_score(1) * attrT[3:4]
    gateT = jnp.clip(1.0 + 0.6 * (angle_score + dihedral_score), 0.35, 2.5)

    theta = (math.pi / CUTOFF) * safeT
    s1 = jnp.sin(theta)
    c2 = 2.0 * jnp.cos(theta)
    inv_clip = 1.0 / safeT
    sines = [s1]
    prev, cur = jnp.zeros_like(s1), s1
    for _ in range(NUM_RADIAL - 1):
        prev, cur = cur, c2 * cur - prev
        sines.append(cur)

    csT = psT[3:4]
    cdT = pdT[3:4]
    scaled_gate = gateT * inv_clip
    featT = jnp.concatenate(
        [s * scaled_gate for s in sines] + [
            distT * (1.0 / CUTOFF) * gateT,
            csT * cdT * gateT,
            jnp.abs(csT - cdT) * gateT,
            jnp.zeros((13,) + gateT.shape[1:], jnp.float32),
        ], axis=0)

    pre = xa_ref[...] + xb_ref[...]
    pre = pre + lax.dot_general(featT, we1s_ref[...], (((0,), (0,)), ((), ())),
                                preferred_element_type=jnp.float32, precision=_PREC)
    pre = pre + be1_ref[...]
    h = _silu(pre)
    mu = jnp.mean(h, axis=1, keepdims=True)
    var = jnp.mean((h - mu) ** 2, axis=1, keepdims=True)
    h = (h - mu) / jnp.sqrt(var + 1e-5) * lng_ref[...] + lnb_ref[...]
    h2 = lax.dot_general(h, we2_ref[...], (((1,), (0,)), ((), ())),
                         preferred_element_type=jnp.float32, precision=_PREC)
    hid = _silu(h2 + be2_ref[...])
    hid_ref[...] = hid

    t = _silu(lax.dot_general(hid, wc1_ref[...], (((1,), (0,)), ((), ())),
                              preferred_element_type=jnp.float32, precision=_PREC)
              + bc1_ref[...])
    coord_w = jnp.sum(t * wc2_ref[...], axis=1, keepdims=True) + bc2_ref[...]
    wvecT = unitT * (coord_w.T * gateT)
    auxT = jnp.concatenate(
        [jnp.ones_like(gateT), wvecT,
         jnp.zeros((12,) + gateT.shape[1:], jnp.float32)], axis=0)
    aux_ref[...] = auxT.T


def _edge_mlp(xa_g, xb_g, ps_g, pd_g, edge_attr, we1s, be1, lng, lnb, we2, be2,
              wc1, bc1, wc2row, bc2, e0, ne):
    nblocks = ne // BE
    blk0 = e0 // BE
    full = lambda shape: pl.BlockSpec(shape, lambda i: (0, 0))
    return pl.pallas_call(
        _edge_body,
        grid=(nblocks,),
        in_specs=[
            pl.BlockSpec((BE, HIDDEN), lambda i: (i, 0)),
            pl.BlockSpec((BE, HIDDEN), lambda i: (i, 0)),
            pl.BlockSpec((BE, 16), lambda i: (i, 0)),
            pl.BlockSpec((BE, 16), lambda i: (i, 0)),
            pl.BlockSpec((BE, 4), lambda i: (i + blk0, 0)),
            full((32, HIDDEN)),
            full((1, HIDDEN)),
            full((1, HIDDEN)),
            full((1, HIDDEN)),
            full((HIDDEN, HIDDEN)),
            full((1, HIDDEN)),
            full((HIDDEN, 64)),
            full((1, 64)),
            full((1, 64)),
            full((1, 1)),
        ],
        out_specs=[
            pl.BlockSpec((BE, HIDDEN), lambda i: (i, 0)),
            pl.BlockSpec((BE, 16), lambda i: (i, 0)),
        ],
        out_shape=[jax.ShapeDtypeStruct((ne, HIDDEN), jnp.float32),
                   jax.ShapeDtypeStruct((ne, 16), jnp.float32)],
    )(xa_g, xb_g, ps_g, pd_g, edge_attr, we1s, be1, lng, lnb, we2, be2,
      wc1, bc1, wc2row, bc2)


def _sc_scatter(hid, aux, src, zeros_h, zeros_a, e0, ne):
    mesh = plsc.VectorSubcoreMesh(core_axis_name="c", subcore_axis_name="s")
    nchunks = ne // CHUNK
    nloop = (nchunks + NW - 1) // NW

    @functools.partial(
        pl.kernel,
        mesh=mesh,
        compiler_params=pltpu.CompilerParams(use_tc_tiling_on_sc=False),
        out_type=[jax.ShapeDtypeStruct((NC, N, HIDDEN), jnp.float32),
                  jax.ShapeDtypeStruct((NC, N, 16), jnp.float32)],
        scratch_types=[pltpu.VMEM((CHUNK,), jnp.int32),
                       pltpu.VMEM((CHUNK, HIDDEN), jnp.float32),
                       pltpu.VMEM((CHUNK, 16), jnp.float32),
                       pltpu.VMEM_SHARED((N, HIDDEN), jnp.float32),
                       pltpu.VMEM_SHARED((N, 16), jnp.float32),
                       pltpu.SemaphoreType.DMA],
    )
    def k(hid_hbm, aux_hbm, src_hbm, zh_hbm, za_hbm, oh_hbm, oa_hbm,
          idx_v, hbuf, abuf, acc_h, acc_a, sem):
        c = lax.axis_index("c")
        s = lax.axis_index("s")

        row0 = s * ROWS_PER_SUB
        pltpu.sync_copy(zh_hbm, acc_h.at[pl.ds(row0, ROWS_PER_SUB)])
        pltpu.sync_copy(za_hbm, acc_a.at[pl.ds(row0, ROWS_PER_SUB)])
        plsc.subcore_barrier()

        @pl.loop(0, nloop)
        def _(j):
            cc = (j * NS + s) * NC + c

            @pl.when(cc < nchunks)
            def _():
                base = cc * CHUNK
                pltpu.sync_copy(src_hbm.at[pl.ds(e0 + base, CHUNK)], idx_v)
                d1 = pltpu.async_copy(hid_hbm.at[pl.ds(base, CHUNK)], hbuf, sem)
                d2 = pltpu.async_copy(aux_hbm.at[pl.ds(base, CHUNK)], abuf, sem)
                d1.wait(); d2.wait()
                pltpu.sync_copy(hbuf, acc_h.at[idx_v], add=True)
                pltpu.sync_copy(abuf, acc_a.at[idx_v], add=True)

        plsc.subcore_barrier()
        pltpu.sync_copy(acc_h.at[pl.ds(row0, ROWS_PER_SUB)],
                        oh_hbm.at[c, pl.ds(row0, ROWS_PER_SUB)])
        pltpu.sync_copy(acc_a.at[pl.ds(row0, ROWS_PER_SUB)],
                        oa_hbm.at[c, pl.ds(row0, ROWS_PER_SUB)])

    return k(hid, aux, src, zeros_h, zeros_a)


def _node_body(nparts, x_ref, pos_ref, *refs):
    (ph_refs, pa_refs) = refs[:nparts], refs[nparts:2 * nparts]
    (wn1_ref, wn2_ref, bn_ref, lng_ref, lnb_ref, xo_ref, po_ref) = refs[2 * nparts:]
    x = x_ref[...]
    hs = sum(r[0] + r[1] for r in ph_refs)
    as_ = sum(r[0] + r[1] for r in pa_refs)
    cnt = jnp.maximum(as_[:, 0:1], 1.0)
    inv = 1.0 / cnt
    agg = hs * inv
    nh = (lax.dot_general(x, wn1_ref[...], (((1,), (0,)), ((), ())),
                          preferred_element_type=jnp.float32, precision=_PREC)
          + lax.dot_general(agg, wn2_ref[...], (((1,), (0,)), ((), ())),
                            preferred_element_type=jnp.float32, precision=_PREC)
          + bn_ref[...])
    nh = _silu(nh)
    mu = jnp.mean(nh, axis=1, keepdims=True)
    var = jnp.mean((nh - mu) ** 2, axis=1, keepdims=True)
    nh = (nh - mu) / jnp.sqrt(var + 1e-5) * lng_ref[...] + lnb_ref[...]
    xo_ref[...] = x + nh
    delta = as_[:, 1:4] * inv
    po_ref[...] = pos_ref[...] + 0.1 * delta


BN = 2000


def _node_update(x, pos, parts_h, parts_a, wn1, wn2, bn, lng, lnb):
    nparts = len(parts_h)
    full = lambda shape: pl.BlockSpec(shape, lambda i: tuple(0 for _ in shape))
    return pl.pallas_call(
        functools.partial(_node_body, nparts),
        grid=(N // BN,),
        in_specs=[
            pl.BlockSpec((BN, HIDDEN), lambda i: (i, 0)),
            pl.BlockSpec((BN, 3), lambda i: (i, 0)),
        ] + [pl.BlockSpec((NC, BN, HIDDEN), lambda i: (0, i, 0))] * nparts
          + [pl.BlockSpec((NC, BN, 16), lambda i: (0, i, 0))] * nparts
          + [
            full((HIDDEN, HIDDEN)),
            full((HIDDEN, HIDDEN)),
            full((1, HIDDEN)),
            full((1, HIDDEN)),
            full((1, HIDDEN)),
        ],
        out_specs=[
            pl.BlockSpec((BN, HIDDEN), lambda i: (i, 0)),
            pl.BlockSpec((BN, 3), lambda i: (i, 0)),
        ],
        out_shape=[jax.ShapeDtypeStruct((N, HIDDEN), jnp.float32),
                   jax.ShapeDtypeStruct((N, 3), jnp.float32)],
    )(x, pos, *parts_h, *parts_a, wn1, wn2, bn, lng, lnb)


def kernel(x, pos, charge, edge_index, edge_attr, W_e1, b_e1, ln_e_g, ln_e_b,
           W_e2, b_e2, W_n, b_n, ln_n_g, ln_n_b, W_c1, b_c1, W_c2, b_c2):
    src = edge_index[0]
    dst = edge_index[1]

    xa, xb = _precompute(x, W_e1[:HIDDEN], W_e1[HIDDEN:2 * HIDDEN])

    small = jnp.concatenate(
        [pos, charge, jnp.zeros((N, 12), jnp.float32)], axis=1)

    we1s = jnp.pad(W_e1[2 * HIDDEN:], ((0, 32 - (W_e1.shape[0] - 2 * HIDDEN)), (0, 0)))
    zeros_h = jnp.zeros((ROWS_PER_SUB, HIDDEN), jnp.float32)
    zeros_a = jnp.zeros((ROWS_PER_SUB, 16), jnp.float32)

    NSPLIT = 4
    EH = E // NSPLIT
    parts = []
    for i in range(NSPLIT):
        e0 = i * EH
        xa_g, xb_g, ps_g, pd_g = _sc_gather(xa, xb, small, src, dst, e0, EH)
        hid, aux = _edge_mlp(
            xa_g, xb_g, ps_g, pd_g, edge_attr,
            we1s, b_e1.reshape(1, HIDDEN), ln_e_g.reshape(1, HIDDEN),
            ln_e_b.reshape(1, HIDDEN), W_e2, b_e2.reshape(1, HIDDEN),
            W_c1, b_c1.reshape(1, 64), W_c2.reshape(1, 64), b_c2.reshape(1, 1),
            e0, EH)
        parts.append(_sc_scatter(hid, aux, src, zeros_h, zeros_a, e0, EH))

    x_new, pos_new = _node_update(
        x, pos, [p[0] for p in parts], [p[1] for p in parts],
        W_n[:HIDDEN], W_n[HIDDEN:],
        b_n.reshape(1, HIDDEN), ln_n_g.reshape(1, HIDDEN), ln_n_b.reshape(1, HIDDEN))
    return (x_new, pos_new)
